# baseline (speedup 1.0000x reference)
import functools

import jax
import jax.numpy as jnp
from jax.experimental import pallas as pl

_ALPHA = 0.5
_BETA = 0.5
_EPS = 1e-06
_C = 81


def _fused_kernel(cp_ref, ct_ref, lp_ref, lt_ref,
                  conf_ref, loc_ref, cnt_ref):
    i = pl.program_id(0)

    ct = ct_ref[...]
    pos = (ct > 0)
    pos_f = pos.astype(jnp.float32)

    p = cp_ref[...]
    lanes = jax.lax.broadcasted_iota(jnp.int32, p.shape, 1)
    is_t = jnp.logical_and(lanes == ct, pos)
    q = jnp.where(is_t, 1.0 - p, p)
    lg = jnp.log(jnp.where(is_t, p, 1.0 - p) + _EPS)
    conf_sum = jnp.sum(q * q * lg, keepdims=True).reshape(1, 1) * (-0.5)

    z = jnp.abs(lp_ref[...] - lt_ref[...])
    sl1 = jnp.where(z < 1.0, 0.5 * z * z, z - 0.5)
    loc_sum = jnp.sum(jnp.sum(sl1, axis=1, keepdims=True) * pos_f,
                      keepdims=True).reshape(1, 1)

    cnt = jnp.sum(pos_f, keepdims=True).reshape(1, 1)

    @pl.when(i == 0)
    def _init():
        conf_ref[...] = conf_sum
        loc_ref[...] = loc_sum
        cnt_ref[...] = cnt

    @pl.when(i != 0)
    def _acc():
        conf_ref[...] += conf_sum
        loc_ref[...] += loc_sum
        cnt_ref[...] += cnt


@functools.partial(jax.jit, static_argnames=())
def _run(loc_preds, loc_targets, conf_preds, conf_targets):
    B, N, C = conf_preds.shape
    M = B * N
    R = 2000
    grid = (M // R,)

    cp = conf_preds.reshape(M, C)
    ct = conf_targets.reshape(M, 1).astype(jnp.int32)
    lp = loc_preds.reshape(M, 4)
    lt = loc_targets.reshape(M, 4)

    out_spec = pl.BlockSpec((1, 1), lambda i: (0, 0))
    conf_sum, loc_sum, cnt = pl.pallas_call(
        _fused_kernel,
        grid=grid,
        in_specs=[
            pl.BlockSpec((R, C), lambda i: (i, 0)),
            pl.BlockSpec((R, 1), lambda i: (i, 0)),
            pl.BlockSpec((R, 4), lambda i: (i, 0)),
            pl.BlockSpec((R, 4), lambda i: (i, 0)),
        ],
        out_specs=[out_spec, out_spec, out_spec],
        out_shape=[
            jax.ShapeDtypeStruct((1, 1), jnp.float32),
            jax.ShapeDtypeStruct((1, 1), jnp.float32),
            jax.ShapeDtypeStruct((1, 1), jnp.float32),
        ],
    )(cp, ct, lp, lt)

    num_matched = cnt[0, 0]
    conf_loss = conf_sum[0, 0] / num_matched
    loc_loss = loc_sum[0, 0] / num_matched
    total = _BETA * conf_loss + (1.0 - _BETA) * loc_loss
    return (total, conf_loss, loc_loss)


def kernel(loc_preds, loc_targets, conf_preds, conf_targets):
    return _run(loc_preds, loc_targets, conf_preds, conf_targets)
